# Initial kernel scaffold; baseline (speedup 1.0000x reference)
import jax
import jax.numpy as jnp
from jax.experimental import pallas as pl


def kernel(arc_scores, arc_to_path, path_to_utt, hyp_tokens, hyp_lens, ref_tokens, ref_lens, num_paths, nbest_scale):
    raise NotImplementedError("write your pallas kernel here")


# speedup vs baseline: 90.0478x; 90.0478x over previous
import functools

import jax
import jax.numpy as jnp
from jax import lax
from jax.experimental import pallas as pl
from jax.experimental.pallas import tpu as pltpu
from jax.experimental.pallas import tpu_sc as plsc

_NUM_PATHS = 3200
_BATCH = 16
_TOTAL_ARCS = 819200
_L_HYP = 128
_L_REF = 128
_NW = 32
_ROWS = _TOTAL_ARCS // _NW // 128


def _sc_segsum(arc_scores, arc_to_path, zeros_init):
    scores2d = arc_scores.reshape(_NW * _ROWS, 128)
    idx2d = arc_to_path.reshape(_NW * _ROWS, 128)
    mesh = plsc.VectorSubcoreMesh(core_axis_name="c", subcore_axis_name="s")

    @functools.partial(
        pl.kernel,
        mesh=mesh,
        out_type=jax.ShapeDtypeStruct((2, _NUM_PATHS), jnp.float32),
        scratch_types=[
            pltpu.VMEM((_ROWS, 128), jnp.float32),
            pltpu.VMEM((_ROWS, 128), jnp.int32),
            pltpu.VMEM_SHARED((_NUM_PATHS,), jnp.float32),
        ],
    )
    def seg_kernel(scores_hbm, idx_hbm, zeros_hbm, out_hbm, vals_v, idx_v, acc_sh):
        c = lax.axis_index("c")
        s = lax.axis_index("s")
        w = c * 16 + s

        @pl.when(s == 0)
        def _zero():
            pltpu.sync_copy(zeros_hbm, acc_sh)

        base = w * _ROWS
        pltpu.sync_copy(scores_hbm.at[pl.ds(base, _ROWS)], vals_v)
        pltpu.sync_copy(idx_hbm.at[pl.ds(base, _ROWS)], idx_v)
        plsc.subcore_barrier()

        def body(j, carry):
            pltpu.sync_copy(vals_v.at[j], acc_sh.at[idx_v.at[j]], add=True)
            return carry

        lax.fori_loop(0, _ROWS, body, 0)
        plsc.subcore_barrier()

        @pl.when(s == 0)
        def _out():
            pltpu.sync_copy(acc_sh, out_hbm.at[c])

    return seg_kernel(scores2d, idx2d, zeros_init)


def _dp_body(partial_ref, hyp_ref, utt_ref, hlen_ref, refT_ref, rlen_ref,
             out_ref, refpp, rlenpp, prev, ans):
    i = pl.program_id(0)
    rowf = (i + 1).astype(jnp.float32)
    P = _NUM_PATHS
    aio_i = lax.broadcasted_iota(jnp.int32, (_L_REF, P), 0) + 1
    aio_f = aio_i.astype(jnp.float32)

    @pl.when(i == 0)
    def _init():
        utt = utt_ref[0:1, :]
        rp = jnp.zeros((_L_REF, P), jnp.int32)
        rl = jnp.zeros((1, P), jnp.int32)
        for u in range(_BATCH):
            m = utt == u
            rp = jnp.where(m, refT_ref[:, u:u + 1], rp)
            rl = jnp.where(m, rlen_ref[0, u], rl)
        refpp[:, :] = rp
        rlenpp[:, :] = rl
        prev[:, :] = aio_f
        ans[:, :] = jnp.zeros((_L_REF, P), jnp.float32)

    hyp_i = hyp_ref[0, 0:1, :]
    cost = jnp.where(refpp[:, :] == hyp_i, 0.0, 1.0).astype(jnp.float32)
    pv = prev[:, :]
    top = jnp.full((1, P), 1.0, jnp.float32) * (rowf - 1.0)
    sh = jnp.concatenate([top, pv[:-1, :]], axis=0)
    cand = jnp.minimum(pv + 1.0, sh + cost)
    t = cand - aio_f
    pm = t
    for sft in (1, 2, 4, 8, 16, 32, 64):
        filler = jnp.full((sft, P), 1e9, jnp.float32)
        pm = jnp.minimum(pm, jnp.concatenate([filler, pm[:-sft, :]], axis=0))
    pm = jnp.minimum(pm, rowf)
    drow = pm + aio_f
    prev[:, :] = drow

    hit = hlen_ref[0:1, :] == (i + 1)
    msk = jnp.logical_and(hit, aio_i == rlenpp[:, :])
    ans[:, :] = jnp.where(msk, drow, ans[:, :])

    @pl.when(i == _L_HYP - 1)
    def _fin():
        wers = jnp.sum(ans[:, :], axis=0, keepdims=True)
        plp = partial_ref[0:1, :] + partial_ref[1:2, :]
        pprob = jnp.exp(plp)
        utt = utt_ref[0:1, :]
        den = jnp.zeros((1, P), jnp.float32)
        for u in range(_BATCH):
            m = utt == u
            du = jnp.sum(jnp.where(m, pprob, 0.0))
            den = jnp.where(m, du, den)
        dlp = jnp.log(den)
        out_ref[:, :] = jnp.sum(jnp.exp(plp - dlp) * wers, axis=1,
                                keepdims=True)


def _tc_mwer(partials, hyp_T3, utt2, hlen2, refT, rlen2):
    P = _NUM_PATHS
    return pl.pallas_call(
        _dp_body,
        grid=(_L_HYP,),
        in_specs=[
            pl.BlockSpec((2, P), lambda i: (0, 0)),
            pl.BlockSpec((1, 1, P), lambda i: (i, 0, 0)),
            pl.BlockSpec((1, P), lambda i: (0, 0)),
            pl.BlockSpec((1, P), lambda i: (0, 0)),
            pl.BlockSpec((_L_REF, _BATCH), lambda i: (0, 0)),
            pl.BlockSpec((1, _BATCH), lambda i: (0, 0)),
        ],
        out_specs=pl.BlockSpec((1, 1), lambda i: (0, 0)),
        out_shape=jax.ShapeDtypeStruct((1, 1), jnp.float32),
        scratch_shapes=[
            pltpu.VMEM((_L_REF, P), jnp.int32),
            pltpu.VMEM((1, P), jnp.int32),
            pltpu.VMEM((_L_REF, P), jnp.float32),
            pltpu.VMEM((_L_REF, P), jnp.float32),
        ],
        compiler_params=pltpu.CompilerParams(
            dimension_semantics=("arbitrary",)),
    )(partials, hyp_T3, utt2, hlen2, refT, rlen2)


def kernel(arc_scores, arc_to_path, path_to_utt, hyp_tokens, hyp_lens,
           ref_tokens, ref_lens, num_paths, nbest_scale):
    del num_paths, nbest_scale
    partials = _sc_segsum(arc_scores, arc_to_path.astype(jnp.int32),
                          jnp.zeros((_NUM_PATHS,), jnp.float32))
    hyp_T3 = hyp_tokens.T.reshape(_L_HYP, 1, _NUM_PATHS)
    out = _tc_mwer(
        partials,
        hyp_T3,
        path_to_utt.reshape(1, _NUM_PATHS),
        hyp_lens.reshape(1, _NUM_PATHS),
        ref_tokens.T,
        ref_lens.reshape(1, _BATCH),
    )
    return out[0, 0]


# speedup vs baseline: 200.4746x; 2.2263x over previous
import functools

import jax
import jax.numpy as jnp
import numpy as np
from jax import lax
from jax.experimental import pallas as pl
from jax.experimental.pallas import tpu as pltpu
from jax.experimental.pallas import tpu_sc as plsc

_NUM_PATHS = 3200
_BATCH = 16
_TOTAL_ARCS = 819200
_L_HYP = 128
_L_REF = 128
_LIMBS = 8
_NW = 32
_ROWS = _TOTAL_ARCS // _NW // 128

_W_np = np.zeros((_LIMBS, _L_REF), np.float32)
for _l in range(_LIMBS):
    for _e in range(16):
        _W_np[_l, 16 * _l + _e] = float(2 ** _e)


def _sc_segsum(arc_scores, arc_to_path, zeros_init):
    scores2d = arc_scores.reshape(_NW * _ROWS, 128)
    idx2d = arc_to_path.reshape(_NW * _ROWS, 128)
    mesh = plsc.VectorSubcoreMesh(core_axis_name="c", subcore_axis_name="s")

    @functools.partial(
        pl.kernel,
        mesh=mesh,
        out_type=jax.ShapeDtypeStruct((2, _NUM_PATHS), jnp.float32),
        scratch_types=[
            pltpu.VMEM((_ROWS, 128), jnp.float32),
            pltpu.VMEM((_ROWS, 128), jnp.int32),
            pltpu.VMEM_SHARED((_NUM_PATHS,), jnp.float32),
        ],
    )
    def seg_kernel(scores_hbm, idx_hbm, zeros_hbm, out_hbm, vals_v, idx_v, acc_sh):
        c = lax.axis_index("c")
        s = lax.axis_index("s")
        w = c * 16 + s

        @pl.when(s == 0)
        def _zero():
            pltpu.sync_copy(zeros_hbm, acc_sh)

        base = w * _ROWS
        pltpu.sync_copy(scores_hbm.at[pl.ds(base, _ROWS)], vals_v)
        pltpu.sync_copy(idx_hbm.at[pl.ds(base, _ROWS)], idx_v)
        plsc.subcore_barrier()

        def body(j, carry):
            pltpu.sync_copy(vals_v.at[j], acc_sh.at[idx_v.at[j]], add=True)
            return carry

        lax.fori_loop(0, _ROWS, body, 0)
        plsc.subcore_barrier()

        @pl.when(s == 0)
        def _out():
            pltpu.sync_copy(acc_sh, out_hbm.at[c])

    return seg_kernel(scores2d, idx2d, zeros_init)


_M16 = 0xFFFF


def _up(x, k):
    return jnp.concatenate(
        [jnp.zeros((k, x.shape[1]), jnp.int32), x[:-k, :]], axis=0)


def _add128(a, b):
    s = a + b
    g = s >> 16
    s = s & _M16
    p = (s + 1) >> 16
    g = g | (p & _up(g, 1))
    p = p & _up(p, 1)
    g = g | (p & _up(g, 2))
    p = p & _up(p, 2)
    g = g | (p & _up(g, 4))
    return (s + _up(g, 1)) & _M16


def _sl1(x):
    return ((x << 1) & _M16) | _up(x >> 15, 1)


def _not16(x):
    return x ^ _M16


def _dp_body(partial_ref, hyp_ref, utt_ref, hlen_ref, refT_ref, rlen_ref,
             w_ref, out_ref, refpp, vp_r, vn_r, score_r, capt_r, maskrl_r,
             c1_r):
    i = pl.program_id(0)
    P = _NUM_PATHS

    @pl.when(i == 0)
    def _init():
        utt = utt_ref[0:1, :]
        rp = jnp.zeros((_L_REF, P), jnp.int32)
        rl = jnp.zeros((1, P), jnp.int32)
        for u in range(_BATCH):
            m = utt == u
            rp = jnp.where(m, refT_ref[:, u:u + 1], rp)
            rl = jnp.where(m, rlen_ref[0, u], rl)
        refpp[:, :] = rp
        liota = lax.broadcasted_iota(jnp.int32, (_LIMBS, P), 0)
        rlm1 = rl - 1
        limb = rlm1 >> 4
        bit = rlm1 & 15
        maskval = jnp.left_shift(jnp.ones_like(bit), bit)
        maskrl_r[:, :] = jnp.where(liota == limb, maskval, 0)
        c1_r[:, :] = jnp.where(liota == 0, 1, 0)
        score_r[:, :] = jnp.where(liota == 0, rl, 0)
        capt_r[:, :] = jnp.zeros((_LIMBS, P), jnp.int32)
        vp_r[:, :] = jnp.full((_LIMBS, P), 0xFFFF, jnp.int32)
        vn_r[:, :] = jnp.zeros((_LIMBS, P), jnp.int32)

    hyp_i = hyp_ref[0, 0:1, :]
    e_f = jnp.where(refpp[:, :] == hyp_i, 1.0, 0.0).astype(jnp.float32)
    eq = lax.dot_general(w_ref[:, :], e_f, (((1,), (0,)), ((), ())),
                         preferred_element_type=jnp.float32
                         ).astype(jnp.int32)

    vp = vp_r[:, :]
    vn = vn_r[:, :]
    xv = eq | vn
    xh = (_add128(eq & vp, vp) ^ vp) | eq
    hp = vn | _not16(xh | vp)
    hn = vp & xh

    mrl = maskrl_r[:, :]
    dplus = jnp.where((hp & mrl) != 0, 1, 0)
    dminus = jnp.where((hn & mrl) != 0, 1, 0)
    score = score_r[:, :] + dplus - dminus
    score_r[:, :] = score
    hit = hlen_ref[0:1, :] == (i + 1)
    capt_r[:, :] = jnp.where(hit, score, capt_r[:, :])

    hps = _sl1(hp) | c1_r[:, :]
    hns = _sl1(hn)
    vp_r[:, :] = hns | _not16(xv | hps)
    vn_r[:, :] = hps & xv

    @pl.when(i == _L_HYP - 1)
    def _fin():
        wers = jnp.sum(capt_r[:, :], axis=0, keepdims=True
                       ).astype(jnp.float32)
        plp = partial_ref[0:1, :] + partial_ref[1:2, :]
        pprob = jnp.exp(plp)
        utt = utt_ref[0:1, :]
        den = jnp.zeros((1, P), jnp.float32)
        for u in range(_BATCH):
            m = utt == u
            du = jnp.sum(jnp.where(m, pprob, 0.0))
            den = jnp.where(m, du, den)
        dlp = jnp.log(den)
        out_ref[:, :] = jnp.sum(jnp.exp(plp - dlp) * wers, axis=1,
                                keepdims=True)


def _tc_mwer(partials, hyp_T3, utt2, hlen2, refT, rlen2, wmat):
    P = _NUM_PATHS
    return pl.pallas_call(
        _dp_body,
        grid=(_L_HYP,),
        in_specs=[
            pl.BlockSpec((2, P), lambda i: (0, 0)),
            pl.BlockSpec((1, 1, P), lambda i: (i, 0, 0)),
            pl.BlockSpec((1, P), lambda i: (0, 0)),
            pl.BlockSpec((1, P), lambda i: (0, 0)),
            pl.BlockSpec((_L_REF, _BATCH), lambda i: (0, 0)),
            pl.BlockSpec((1, _BATCH), lambda i: (0, 0)),
            pl.BlockSpec((_LIMBS, _L_REF), lambda i: (0, 0)),
        ],
        out_specs=pl.BlockSpec((1, 1), lambda i: (0, 0)),
        out_shape=jax.ShapeDtypeStruct((1, 1), jnp.float32),
        scratch_shapes=[
            pltpu.VMEM((_L_REF, P), jnp.int32),
            pltpu.VMEM((_LIMBS, P), jnp.int32),
            pltpu.VMEM((_LIMBS, P), jnp.int32),
            pltpu.VMEM((_LIMBS, P), jnp.int32),
            pltpu.VMEM((_LIMBS, P), jnp.int32),
            pltpu.VMEM((_LIMBS, P), jnp.int32),
            pltpu.VMEM((_LIMBS, P), jnp.int32),
        ],
        compiler_params=pltpu.CompilerParams(
            dimension_semantics=("arbitrary",)),
    )(partials, hyp_T3, utt2, hlen2, refT, rlen2, wmat)


def kernel(arc_scores, arc_to_path, path_to_utt, hyp_tokens, hyp_lens,
           ref_tokens, ref_lens, num_paths, nbest_scale):
    del num_paths, nbest_scale
    partials = _sc_segsum(arc_scores, arc_to_path.astype(jnp.int32),
                          jnp.zeros((_NUM_PATHS,), jnp.float32))
    hyp_T3 = hyp_tokens.T.reshape(_L_HYP, 1, _NUM_PATHS)
    out = _tc_mwer(
        partials,
        hyp_T3,
        path_to_utt.reshape(1, _NUM_PATHS),
        hyp_lens.reshape(1, _NUM_PATHS),
        ref_tokens.T,
        ref_lens.reshape(1, _BATCH),
        jnp.asarray(_W_np),
    )
    return out[0, 0]


# speedup vs baseline: 255.5180x; 1.2746x over previous
import functools

import jax
import jax.numpy as jnp
import numpy as np
from jax import lax
from jax.experimental import pallas as pl
from jax.experimental.pallas import tpu as pltpu
from jax.experimental.pallas import tpu_sc as plsc

_NUM_PATHS = 3200
_BATCH = 16
_TOTAL_ARCS = 819200
_L_HYP = 128
_L_REF = 128
_LIMBS = 8
_NW = 32
_ROWS = _TOTAL_ARCS // _NW // 128

_W_np = np.zeros((_LIMBS, _L_REF), np.float32)
for _l in range(_LIMBS):
    for _e in range(16):
        _W_np[_l, 16 * _l + _e] = float(2 ** _e)


def _sc_segsum(arc_scores, arc_to_path, zeros_init):
    scores2d = arc_scores.reshape(_NW * _ROWS, 128)
    idx2d = arc_to_path.reshape(_NW * _ROWS, 128)
    mesh = plsc.VectorSubcoreMesh(core_axis_name="c", subcore_axis_name="s")

    @functools.partial(
        pl.kernel,
        mesh=mesh,
        out_type=jax.ShapeDtypeStruct((2, _NUM_PATHS), jnp.float32),
        scratch_types=[
            pltpu.VMEM((_ROWS, 128), jnp.float32),
            pltpu.VMEM((_ROWS, 128), jnp.int32),
            pltpu.VMEM_SHARED((_NUM_PATHS,), jnp.float32),
        ],
    )
    def seg_kernel(scores_hbm, idx_hbm, zeros_hbm, out_hbm, vals_v, idx_v, acc_sh):
        c = lax.axis_index("c")
        s = lax.axis_index("s")
        w = c * 16 + s

        @pl.when(s == 0)
        def _zero():
            pltpu.sync_copy(zeros_hbm, acc_sh)

        base = w * _ROWS
        pltpu.sync_copy(scores_hbm.at[pl.ds(base, _ROWS)], vals_v)
        pltpu.sync_copy(idx_hbm.at[pl.ds(base, _ROWS)], idx_v)
        plsc.subcore_barrier()

        def body(j, carry):
            pltpu.sync_copy(vals_v.at[j], acc_sh.at[idx_v.at[j]], add=True)
            return carry

        lax.fori_loop(0, _ROWS, body, 0)
        plsc.subcore_barrier()

        @pl.when(s == 0)
        def _out():
            pltpu.sync_copy(acc_sh, out_hbm.at[c])

    return seg_kernel(scores2d, idx2d, zeros_init)


_M16 = 0xFFFF


def _up(x, k):
    return jnp.concatenate(
        [jnp.zeros((k, x.shape[1]), jnp.int32), x[:-k, :]], axis=0)


def _add128(a, b):
    s = a + b
    g = s >> 16
    s = s & _M16
    p = (s + 1) >> 16
    g = g | (p & _up(g, 1))
    p = p & _up(p, 1)
    g = g | (p & _up(g, 2))
    p = p & _up(p, 2)
    g = g | (p & _up(g, 4))
    return (s + _up(g, 1)) & _M16


def _sl1(x):
    return ((x << 1) & _M16) | _up(x >> 15, 1)


def _not16(x):
    return x ^ _M16


def _dp_body(hyp_ref, utt_ref, hlen_ref, refT_ref, rlen_ref,
             w_ref, out_ref, refpp, vp_r, vn_r, score_r, capt_r, maskrl_r,
             c1_r):
    i = pl.program_id(0)
    P = _NUM_PATHS

    @pl.when(i == 0)
    def _init():
        utt = utt_ref[0:1, :]
        rp = jnp.zeros((_L_REF, P), jnp.int32)
        rl = jnp.zeros((1, P), jnp.int32)
        for u in range(_BATCH):
            m = utt == u
            rp = jnp.where(m, refT_ref[:, u:u + 1], rp)
            rl = jnp.where(m, rlen_ref[0, u], rl)
        refpp[:, :] = rp
        liota = lax.broadcasted_iota(jnp.int32, (_LIMBS, P), 0)
        rlm1 = rl - 1
        limb = rlm1 >> 4
        bit = rlm1 & 15
        maskval = jnp.left_shift(jnp.ones_like(bit), bit)
        maskrl_r[:, :] = jnp.where(liota == limb, maskval, 0)
        c1_r[:, :] = jnp.where(liota == 0, 1, 0)
        score_r[:, :] = jnp.where(liota == 0, rl, 0)
        capt_r[:, :] = jnp.zeros((_LIMBS, P), jnp.int32)
        vp_r[:, :] = jnp.full((_LIMBS, P), 0xFFFF, jnp.int32)
        vn_r[:, :] = jnp.zeros((_LIMBS, P), jnp.int32)

    hyp_i = hyp_ref[0, 0:1, :]
    e_f = jnp.where(refpp[:, :] == hyp_i, 1.0, 0.0).astype(jnp.float32)
    eq = lax.dot_general(w_ref[:, :], e_f, (((1,), (0,)), ((), ())),
                         preferred_element_type=jnp.float32
                         ).astype(jnp.int32)

    vp = vp_r[:, :]
    vn = vn_r[:, :]
    xv = eq | vn
    xh = (_add128(eq & vp, vp) ^ vp) | eq
    hp = vn | _not16(xh | vp)
    hn = vp & xh

    mrl = maskrl_r[:, :]
    dplus = jnp.where((hp & mrl) != 0, 1, 0)
    dminus = jnp.where((hn & mrl) != 0, 1, 0)
    score = score_r[:, :] + dplus - dminus
    score_r[:, :] = score
    hit = hlen_ref[0:1, :] == (i + 1)
    capt_r[:, :] = jnp.where(hit, score, capt_r[:, :])

    hps = _sl1(hp) | c1_r[:, :]
    hns = _sl1(hn)
    vp_r[:, :] = hns | _not16(xv | hps)
    vn_r[:, :] = hps & xv

    @pl.when(i == _L_HYP - 1)
    def _fin():
        out_ref[:, :] = jnp.sum(capt_r[:, :], axis=0, keepdims=True
                                ).astype(jnp.float32)


def _final_body(partial_ref, wers_ref, utt_ref, out_ref):
    P = _NUM_PATHS
    wers = wers_ref[:, :]
    plp = partial_ref[0:1, :] + partial_ref[1:2, :]
    pprob = jnp.exp(plp)
    utt = utt_ref[:, :]
    den = jnp.zeros((1, P), jnp.float32)
    for u in range(_BATCH):
        m = utt == u
        du = jnp.sum(jnp.where(m, pprob, 0.0))
        den = jnp.where(m, du, den)
    dlp = jnp.log(den)
    out_ref[:, :] = jnp.sum(jnp.exp(plp - dlp) * wers, axis=1,
                            keepdims=True)


def _tc_final(partials, wers, utt2):
    P = _NUM_PATHS
    return pl.pallas_call(
        _final_body,
        out_shape=jax.ShapeDtypeStruct((1, 1), jnp.float32),
    )(partials, wers, utt2)


def _tc_mwer(hyp_T3, utt2, hlen2, refT, rlen2, wmat):
    P = _NUM_PATHS
    return pl.pallas_call(
        _dp_body,
        grid=(_L_HYP,),
        in_specs=[
            pl.BlockSpec((1, 1, P), lambda i: (i, 0, 0)),
            pl.BlockSpec((1, P), lambda i: (0, 0)),
            pl.BlockSpec((1, P), lambda i: (0, 0)),
            pl.BlockSpec((_L_REF, _BATCH), lambda i: (0, 0)),
            pl.BlockSpec((1, _BATCH), lambda i: (0, 0)),
            pl.BlockSpec((_LIMBS, _L_REF), lambda i: (0, 0)),
        ],
        out_specs=pl.BlockSpec((1, P), lambda i: (0, 0)),
        out_shape=jax.ShapeDtypeStruct((1, P), jnp.float32),
        scratch_shapes=[
            pltpu.VMEM((_L_REF, P), jnp.int32),
            pltpu.VMEM((_LIMBS, P), jnp.int32),
            pltpu.VMEM((_LIMBS, P), jnp.int32),
            pltpu.VMEM((_LIMBS, P), jnp.int32),
            pltpu.VMEM((_LIMBS, P), jnp.int32),
            pltpu.VMEM((_LIMBS, P), jnp.int32),
            pltpu.VMEM((_LIMBS, P), jnp.int32),
        ],
        compiler_params=pltpu.CompilerParams(
            dimension_semantics=("arbitrary",)),
    )(hyp_T3, utt2, hlen2, refT, rlen2, wmat)


def kernel(arc_scores, arc_to_path, path_to_utt, hyp_tokens, hyp_lens,
           ref_tokens, ref_lens, num_paths, nbest_scale):
    del num_paths, nbest_scale
    partials = _sc_segsum(arc_scores, arc_to_path.astype(jnp.int32),
                          jnp.zeros((_NUM_PATHS,), jnp.float32))
    hyp_T3 = hyp_tokens.T.reshape(_L_HYP, 1, _NUM_PATHS)
    utt2 = path_to_utt.reshape(1, _NUM_PATHS)
    wers = _tc_mwer(
        hyp_T3,
        utt2,
        hyp_lens.reshape(1, _NUM_PATHS),
        ref_tokens.T,
        ref_lens.reshape(1, _BATCH),
        jnp.asarray(_W_np),
    )
    out = _tc_final(partials, wers, utt2)
    return out[0, 0]


# speedup vs baseline: 386.0195x; 1.5107x over previous
import functools

import jax
import jax.numpy as jnp
import numpy as np
from jax import lax
from jax.experimental import pallas as pl
from jax.experimental.pallas import tpu as pltpu
from jax.experimental.pallas import tpu_sc as plsc

_NUM_PATHS = 3200
_BATCH = 16
_TOTAL_ARCS = 819200
_L_HYP = 128
_L_REF = 128
_LIMBS = 8
_NW = 32
_ROWS = _TOTAL_ARCS // _NW // 128

_W_np = np.zeros((_LIMBS, _L_REF), np.float32)
for _l in range(_LIMBS):
    for _e in range(16):
        _W_np[_l, 16 * _l + _e] = float(2 ** _e)


def _sc_segsum(arc_scores, arc_to_path, zeros_init):
    scores2d = arc_scores.reshape(_NW * _ROWS, 128)
    idx2d = arc_to_path.reshape(_NW * _ROWS, 128)
    mesh = plsc.VectorSubcoreMesh(core_axis_name="c", subcore_axis_name="s")

    @functools.partial(
        pl.kernel,
        mesh=mesh,
        out_type=jax.ShapeDtypeStruct((2, _NUM_PATHS), jnp.float32),
        scratch_types=[
            pltpu.VMEM((_ROWS, 128), jnp.float32),
            pltpu.VMEM((_ROWS, 128), jnp.int32),
            pltpu.VMEM_SHARED((_NUM_PATHS,), jnp.float32),
        ],
    )
    def seg_kernel(scores_hbm, idx_hbm, zeros_hbm, out_hbm, vals_v, idx_v, acc_sh):
        c = lax.axis_index("c")
        s = lax.axis_index("s")
        w = c * 16 + s

        @pl.when(s == 0)
        def _zero():
            pltpu.sync_copy(zeros_hbm, acc_sh)

        base = w * _ROWS
        pltpu.sync_copy(scores_hbm.at[pl.ds(base, _ROWS)], vals_v)
        pltpu.sync_copy(idx_hbm.at[pl.ds(base, _ROWS)], idx_v)
        plsc.subcore_barrier()

        def body(j, carry):
            pltpu.sync_copy(vals_v.at[j], acc_sh.at[idx_v.at[j]], add=True)
            return carry

        lax.fori_loop(0, _ROWS, body, 0)
        plsc.subcore_barrier()

        @pl.when(s == 0)
        def _out():
            pltpu.sync_copy(acc_sh, out_hbm.at[c])

    return seg_kernel(scores2d, idx2d, zeros_init)


_M16 = 0xFFFF


def _up(x, k):
    return jnp.concatenate(
        [jnp.zeros((k, x.shape[1]), jnp.int32), x[:-k, :]], axis=0)


def _add128(a, b):
    s = a + b
    g = s >> 16
    s = s & _M16
    p = (s + 1) >> 16
    g = g | (p & _up(g, 1))
    p = p & _up(p, 1)
    g = g | (p & _up(g, 2))
    p = p & _up(p, 2)
    g = g | (p & _up(g, 4))
    return (s + _up(g, 1)) & _M16


def _sl1(x):
    return ((x << 1) & _M16) | _up(x >> 15, 1)


def _not16(x):
    return x ^ _M16


_CPB = 4


def _dp_body(hyp_ref, utt_ref, hlen_ref, refT_ref, rlen_ref,
             w_ref, out_ref, refpp, vp_r, vn_r, score_r, capt_r, maskrl_r,
             c1_r):
    i = pl.program_id(0)
    P = _NUM_PATHS

    @pl.when(i == 0)
    def _init():
        utt = utt_ref[0:1, :]
        rp = jnp.zeros((_L_REF, P), jnp.int32)
        rl = jnp.zeros((1, P), jnp.int32)
        for u in range(_BATCH):
            m = utt == u
            rp = jnp.where(m, refT_ref[:, u:u + 1], rp)
            rl = jnp.where(m, rlen_ref[0, u], rl)
        refpp[:, :] = rp
        liota = lax.broadcasted_iota(jnp.int32, (_LIMBS, P), 0)
        rlm1 = rl - 1
        limb = rlm1 >> 4
        bit = rlm1 & 15
        maskval = jnp.left_shift(jnp.ones_like(bit), bit)
        maskrl_r[:, :] = jnp.where(liota == limb, maskval, 0)
        c1_r[:, :] = jnp.where(liota == 0, 1, 0)
        score_r[:, :] = jnp.where(liota == 0, rl, 0)
        capt_r[:, :] = jnp.zeros((_LIMBS, P), jnp.int32)
        vp_r[:, :] = jnp.full((_LIMBS, P), 0xFFFF, jnp.int32)
        vn_r[:, :] = jnp.zeros((_LIMBS, P), jnp.int32)

    vp = vp_r[:, :]
    vn = vn_r[:, :]
    score = score_r[:, :]
    capt = capt_r[:, :]
    mrl = maskrl_r[:, :]
    c1 = c1_r[:, :]
    rp = refpp[:, :]
    w = w_ref[:, :]
    hlen = hlen_ref[0:1, :]

    for k in range(_CPB):
        hyp_i = hyp_ref[k, 0:1, :]
        e_f = jnp.where(rp == hyp_i, 1.0, 0.0).astype(jnp.float32)
        eq = lax.dot_general(w, e_f, (((1,), (0,)), ((), ())),
                             preferred_element_type=jnp.float32
                             ).astype(jnp.int32)

        xv = eq | vn
        xh = (_add128(eq & vp, vp) ^ vp) | eq
        hp = vn | _not16(xh | vp)
        hn = vp & xh

        dplus = jnp.where((hp & mrl) != 0, 1, 0)
        dminus = jnp.where((hn & mrl) != 0, 1, 0)
        score = score + dplus - dminus
        hit = hlen == (i * _CPB + k + 1)
        capt = jnp.where(hit, score, capt)

        hps = _sl1(hp) | c1
        hns = _sl1(hn)
        vp = hns | _not16(xv | hps)
        vn = hps & xv

    vp_r[:, :] = vp
    vn_r[:, :] = vn
    score_r[:, :] = score
    capt_r[:, :] = capt

    @pl.when(i == _L_HYP // _CPB - 1)
    def _fin():
        out_ref[:, :] = jnp.sum(capt, axis=0, keepdims=True
                                ).astype(jnp.float32)


def _final_body(partial_ref, wers_ref, utt_ref, out_ref):
    P = _NUM_PATHS
    wers = wers_ref[:, :]
    plp = partial_ref[0:1, :] + partial_ref[1:2, :]
    pprob = jnp.exp(plp)
    utt = utt_ref[:, :]
    den = jnp.zeros((1, P), jnp.float32)
    for u in range(_BATCH):
        m = utt == u
        du = jnp.sum(jnp.where(m, pprob, 0.0))
        den = jnp.where(m, du, den)
    dlp = jnp.log(den)
    out_ref[:, :] = jnp.sum(jnp.exp(plp - dlp) * wers, axis=1,
                            keepdims=True)


def _tc_final(partials, wers, utt2):
    P = _NUM_PATHS
    return pl.pallas_call(
        _final_body,
        out_shape=jax.ShapeDtypeStruct((1, 1), jnp.float32),
    )(partials, wers, utt2)


def _tc_mwer(hyp_T3, utt2, hlen2, refT, rlen2, wmat):
    P = _NUM_PATHS
    return pl.pallas_call(
        _dp_body,
        grid=(_L_HYP // _CPB,),
        in_specs=[
            pl.BlockSpec((_CPB, 1, P), lambda i: (i, 0, 0)),
            pl.BlockSpec((1, P), lambda i: (0, 0)),
            pl.BlockSpec((1, P), lambda i: (0, 0)),
            pl.BlockSpec((_L_REF, _BATCH), lambda i: (0, 0)),
            pl.BlockSpec((1, _BATCH), lambda i: (0, 0)),
            pl.BlockSpec((_LIMBS, _L_REF), lambda i: (0, 0)),
        ],
        out_specs=pl.BlockSpec((1, P), lambda i: (0, 0)),
        out_shape=jax.ShapeDtypeStruct((1, P), jnp.float32),
        scratch_shapes=[
            pltpu.VMEM((_L_REF, P), jnp.int32),
            pltpu.VMEM((_LIMBS, P), jnp.int32),
            pltpu.VMEM((_LIMBS, P), jnp.int32),
            pltpu.VMEM((_LIMBS, P), jnp.int32),
            pltpu.VMEM((_LIMBS, P), jnp.int32),
            pltpu.VMEM((_LIMBS, P), jnp.int32),
            pltpu.VMEM((_LIMBS, P), jnp.int32),
        ],
        compiler_params=pltpu.CompilerParams(
            dimension_semantics=("arbitrary",)),
    )(hyp_T3, utt2, hlen2, refT, rlen2, wmat)


def kernel(arc_scores, arc_to_path, path_to_utt, hyp_tokens, hyp_lens,
           ref_tokens, ref_lens, num_paths, nbest_scale):
    del num_paths, nbest_scale
    partials = _sc_segsum(arc_scores, arc_to_path.astype(jnp.int32),
                          jnp.zeros((_NUM_PATHS,), jnp.float32))
    hyp_T3 = hyp_tokens.T.reshape(_L_HYP, 1, _NUM_PATHS)
    utt2 = path_to_utt.reshape(1, _NUM_PATHS)
    wers = _tc_mwer(
        hyp_T3,
        utt2,
        hyp_lens.reshape(1, _NUM_PATHS),
        ref_tokens.T,
        ref_lens.reshape(1, _BATCH),
        jnp.asarray(_W_np),
    )
    out = _tc_final(partials, wers, utt2)
    return out[0, 0]


# speedup vs baseline: 396.0293x; 1.0259x over previous
import functools

import jax
import jax.numpy as jnp
import numpy as np
from jax import lax
from jax.experimental import pallas as pl
from jax.experimental.pallas import tpu as pltpu
from jax.experimental.pallas import tpu_sc as plsc

_NUM_PATHS = 3200
_BATCH = 16
_TOTAL_ARCS = 819200
_L_HYP = 128
_L_REF = 128
_LIMBS = 8
_NW = 32
_ROWS = _TOTAL_ARCS // _NW // 128

_W_np = np.zeros((_LIMBS, _L_REF), np.float32)
for _l in range(_LIMBS):
    for _e in range(16):
        _W_np[_l, 16 * _l + _e] = float(2 ** _e)


def _sc_segsum(arc_scores, arc_to_path, zeros_init):
    scores2d = arc_scores.reshape(_NW * _ROWS, 128)
    idx2d = arc_to_path.reshape(_NW * _ROWS, 128)
    mesh = plsc.VectorSubcoreMesh(core_axis_name="c", subcore_axis_name="s")

    @functools.partial(
        pl.kernel,
        mesh=mesh,
        out_type=jax.ShapeDtypeStruct((2, _NUM_PATHS), jnp.float32),
        scratch_types=[
            pltpu.VMEM((_ROWS, 128), jnp.float32),
            pltpu.VMEM((_ROWS, 128), jnp.int32),
            pltpu.VMEM_SHARED((_NUM_PATHS,), jnp.float32),
        ],
    )
    def seg_kernel(scores_hbm, idx_hbm, zeros_hbm, out_hbm, vals_v, idx_v, acc_sh):
        c = lax.axis_index("c")
        s = lax.axis_index("s")
        w = c * 16 + s

        @pl.when(s == 0)
        def _zero():
            pltpu.sync_copy(zeros_hbm, acc_sh)

        base = w * _ROWS
        pltpu.sync_copy(scores_hbm.at[pl.ds(base, _ROWS)], vals_v)
        pltpu.sync_copy(idx_hbm.at[pl.ds(base, _ROWS)], idx_v)
        plsc.subcore_barrier()

        def body(j, carry):
            pltpu.sync_copy(vals_v.at[j], acc_sh.at[idx_v.at[j]], add=True)
            return carry

        lax.fori_loop(0, _ROWS, body, 0)
        plsc.subcore_barrier()

        @pl.when(s == 0)
        def _out():
            pltpu.sync_copy(acc_sh, out_hbm.at[c])

    return seg_kernel(scores2d, idx2d, zeros_init)


_M16 = 0xFFFF


def _up(x, k):
    return jnp.concatenate(
        [jnp.zeros((k, x.shape[1]), jnp.int32), x[:-k, :]], axis=0)


def _add128(a, b):
    s = a + b
    g = s >> 16
    s = s & _M16
    p = (s + 1) >> 16
    g = g | (p & _up(g, 1))
    p = p & _up(p, 1)
    g = g | (p & _up(g, 2))
    p = p & _up(p, 2)
    g = g | (p & _up(g, 4))
    return (s + _up(g, 1)) & _M16


def _sl1(x):
    return ((x << 1) & _M16) | _up(x >> 15, 1)


def _not16(x):
    return x ^ _M16


_CPB = 8


def _dp_body(hyp_ref, utt_ref, hlen_ref, refT_ref, rlen_ref,
             w_ref, out_ref, refpp, vp_r, vn_r, score_r, capt_r, maskrl_r,
             c1_r):
    i = pl.program_id(0)
    P = _NUM_PATHS

    @pl.when(i == 0)
    def _init():
        utt = utt_ref[0:1, :]
        rp = jnp.zeros((_L_REF, P), jnp.int32)
        rl = jnp.zeros((1, P), jnp.int32)
        for u in range(_BATCH):
            m = utt == u
            rp = jnp.where(m, refT_ref[:, u:u + 1], rp)
            rl = jnp.where(m, rlen_ref[0, u], rl)
        refpp[:, :] = rp
        liota = lax.broadcasted_iota(jnp.int32, (_LIMBS, P), 0)
        rlm1 = rl - 1
        limb = rlm1 >> 4
        bit = rlm1 & 15
        maskval = jnp.left_shift(jnp.ones_like(bit), bit)
        maskrl_r[:, :] = jnp.where(liota == limb, maskval, 0)
        c1_r[:, :] = jnp.where(liota == 0, 1, 0)
        score_r[:, :] = jnp.where(liota == 0, rl, 0)
        capt_r[:, :] = jnp.zeros((_LIMBS, P), jnp.int32)
        vp_r[:, :] = jnp.full((_LIMBS, P), 0xFFFF, jnp.int32)
        vn_r[:, :] = jnp.zeros((_LIMBS, P), jnp.int32)

    vp = vp_r[:, :]
    vn = vn_r[:, :]
    score = score_r[:, :]
    capt = capt_r[:, :]
    mrl = maskrl_r[:, :]
    c1 = c1_r[:, :]
    rp = refpp[:, :]
    w = w_ref[:, :]
    hlen = hlen_ref[0:1, :]

    for k in range(_CPB):
        hyp_i = hyp_ref[k, 0:1, :]
        e_f = jnp.where(rp == hyp_i, 1.0, 0.0).astype(jnp.float32)
        eq = lax.dot_general(w, e_f, (((1,), (0,)), ((), ())),
                             preferred_element_type=jnp.float32
                             ).astype(jnp.int32)

        xv = eq | vn
        xh = (_add128(eq & vp, vp) ^ vp) | eq
        hp = vn | _not16(xh | vp)
        hn = vp & xh

        dplus = jnp.where((hp & mrl) != 0, 1, 0)
        dminus = jnp.where((hn & mrl) != 0, 1, 0)
        score = score + dplus - dminus
        hit = hlen == (i * _CPB + k + 1)
        capt = jnp.where(hit, score, capt)

        hps = _sl1(hp) | c1
        hns = _sl1(hn)
        vp = hns | _not16(xv | hps)
        vn = hps & xv

    vp_r[:, :] = vp
    vn_r[:, :] = vn
    score_r[:, :] = score
    capt_r[:, :] = capt

    @pl.when(i == _L_HYP // _CPB - 1)
    def _fin():
        out_ref[:, :] = jnp.sum(capt, axis=0, keepdims=True
                                ).astype(jnp.float32)


def _final_body(partial_ref, wers_ref, utt_ref, out_ref):
    P = _NUM_PATHS
    wers = wers_ref[:, :]
    plp = partial_ref[0:1, :] + partial_ref[1:2, :]
    pprob = jnp.exp(plp)
    utt = utt_ref[:, :]
    den = jnp.zeros((1, P), jnp.float32)
    for u in range(_BATCH):
        m = utt == u
        du = jnp.sum(jnp.where(m, pprob, 0.0))
        den = jnp.where(m, du, den)
    dlp = jnp.log(den)
    out_ref[:, :] = jnp.sum(jnp.exp(plp - dlp) * wers, axis=1,
                            keepdims=True)


def _tc_final(partials, wers, utt2):
    P = _NUM_PATHS
    return pl.pallas_call(
        _final_body,
        out_shape=jax.ShapeDtypeStruct((1, 1), jnp.float32),
    )(partials, wers, utt2)


def _tc_mwer(hyp_T3, utt2, hlen2, refT, rlen2, wmat):
    P = _NUM_PATHS
    return pl.pallas_call(
        _dp_body,
        grid=(_L_HYP // _CPB,),
        in_specs=[
            pl.BlockSpec((_CPB, 1, P), lambda i: (i, 0, 0)),
            pl.BlockSpec((1, P), lambda i: (0, 0)),
            pl.BlockSpec((1, P), lambda i: (0, 0)),
            pl.BlockSpec((_L_REF, _BATCH), lambda i: (0, 0)),
            pl.BlockSpec((1, _BATCH), lambda i: (0, 0)),
            pl.BlockSpec((_LIMBS, _L_REF), lambda i: (0, 0)),
        ],
        out_specs=pl.BlockSpec((1, P), lambda i: (0, 0)),
        out_shape=jax.ShapeDtypeStruct((1, P), jnp.float32),
        scratch_shapes=[
            pltpu.VMEM((_L_REF, P), jnp.int32),
            pltpu.VMEM((_LIMBS, P), jnp.int32),
            pltpu.VMEM((_LIMBS, P), jnp.int32),
            pltpu.VMEM((_LIMBS, P), jnp.int32),
            pltpu.VMEM((_LIMBS, P), jnp.int32),
            pltpu.VMEM((_LIMBS, P), jnp.int32),
            pltpu.VMEM((_LIMBS, P), jnp.int32),
        ],
        compiler_params=pltpu.CompilerParams(
            dimension_semantics=("arbitrary",)),
    )(hyp_T3, utt2, hlen2, refT, rlen2, wmat)


def kernel(arc_scores, arc_to_path, path_to_utt, hyp_tokens, hyp_lens,
           ref_tokens, ref_lens, num_paths, nbest_scale):
    del num_paths, nbest_scale
    partials = _sc_segsum(arc_scores, arc_to_path.astype(jnp.int32),
                          jnp.zeros((_NUM_PATHS,), jnp.float32))
    hyp_T3 = hyp_tokens.T.reshape(_L_HYP, 1, _NUM_PATHS)
    utt2 = path_to_utt.reshape(1, _NUM_PATHS)
    wers = _tc_mwer(
        hyp_T3,
        utt2,
        hyp_lens.reshape(1, _NUM_PATHS),
        ref_tokens.T,
        ref_lens.reshape(1, _BATCH),
        jnp.asarray(_W_np),
    )
    out = _tc_final(partials, wers, utt2)
    return out[0, 0]
